# baseline (speedup 1.0000x reference)
import functools

import numpy as np
import jax
import jax.numpy as jnp
from jax import lax
from jax.experimental import pallas as pl
from jax.experimental.pallas import tpu as pltpu
from jax.experimental.pallas import tpu_sc as plsc

N = 10000
NP = 10240
E = 320000
EPAD = 327680
D_IN = 128
D_H = 256
D_OUT = 64

NCORE = 2
NTILE = 16
ROWS_PT = NP // NTILE

CHD = 128
EW = EPAD // (NCORE * NTILE)
CW = EW // CHD

CH = 128
EPT = EPAD // NTILE
CT = EPT // CH
GC = 16
G = CT // GC

BR = 1024
GR = NP // BR

_BN_S = float(1.0 / np.sqrt(1.0 + 1e-5))

_MESH = plsc.VectorSubcoreMesh(
    core_axis_name="c", subcore_axis_name="s", num_cores=NCORE,
    num_subcores=NTILE)


@functools.partial(
    pl.kernel,
    out_type=jax.ShapeDtypeStruct((NCORE, NP), jnp.float32),
    mesh=_MESH,
    scratch_types=[
        pltpu.VMEM((CW, CHD), jnp.int32),
        pltpu.VMEM((CHD,), jnp.float32),
        pltpu.VMEM((ROWS_PT,), jnp.float32),
        pltpu.VMEM_SHARED((NP,), jnp.float32),
        pltpu.SemaphoreType.DMA,
    ],
)
def _deg_kernel(dst_hbm, deg_out, dst_v, ones_v, zb_v, acc_sh, sem):
    c = lax.axis_index("c")
    s = lax.axis_index("s")

    def fz(i, _):
        zb_v[pl.ds(i * 16, 16)] = jnp.zeros((16,), jnp.float32)
        return 0
    lax.fori_loop(0, ROWS_PT // 16, fz, 0)

    def fo(i, _):
        ones_v[pl.ds(i * 16, 16)] = jnp.ones((16,), jnp.float32)
        return 0
    lax.fori_loop(0, CHD // 16, fo, 0)

    pltpu.sync_copy(dst_hbm.at[c, s], dst_v)
    pltpu.sync_copy(zb_v, acc_sh.at[pl.ds(s * ROWS_PT, ROWS_PT)])
    plsc.subcore_barrier()

    def body(i, _):
        pltpu.async_copy(ones_v, acc_sh.at[dst_v.at[i]], sem, add=True)

        @pl.when(i >= 8)
        def _():
            pltpu.make_async_copy(ones_v, acc_sh.at[dst_v.at[i]], sem).wait()
        return 0
    lax.fori_loop(0, CW, body, 0)

    def drain(i, _):
        pltpu.make_async_copy(ones_v, acc_sh.at[dst_v.at[0]], sem).wait()
        return 0
    lax.fori_loop(0, 8, drain, 0)

    plsc.subcore_barrier()
    pltpu.sync_copy(acc_sh.at[pl.ds(s * ROWS_PT, ROWS_PT)],
                    deg_out.at[c, pl.ds(s * ROWS_PT, ROWS_PT)])


def _make_agg(d_half):
    @functools.partial(
        pl.kernel,
        out_type=jax.ShapeDtypeStruct((NCORE, NP, d_half), jnp.bfloat16),
        mesh=_MESH,
        compiler_params=pltpu.CompilerParams(use_tc_tiling_on_sc=False),
        scratch_types=[
            pltpu.VMEM((2, GC, CH), jnp.int32),
            pltpu.VMEM((2, GC, CH), jnp.int32),
            pltpu.VMEM((CH, d_half), jnp.bfloat16),
            pltpu.VMEM((CH, d_half), jnp.bfloat16),
            pltpu.VMEM_SHARED((NP, d_half), jnp.bfloat16),
            pltpu.SemaphoreType.DMA,
            pltpu.SemaphoreType.DMA,
            pltpu.SemaphoreType.DMA,
            pltpu.SemaphoreType.DMA,
            pltpu.SemaphoreType.DMA,
        ],
    )
    def agg(y_hbm, src2_hbm, dst_hbm, z_hbm, src_v, dst_v, rows0, rows1,
            acc_sh, si, sg0, sg1, ss0, ss1):
        c = lax.axis_index("c")
        s = lax.axis_index("s")
        yc = y_hbm.at[c]
        pltpu.sync_copy(src2_hbm.at[s, 0], src_v.at[0])
        pltpu.sync_copy(dst_hbm.at[s, 0], dst_v.at[0])
        pltpu.sync_copy(yc.at[pl.ds(s * ROWS_PT, ROWS_PT)],
                        acc_sh.at[pl.ds(s * ROWS_PT, ROWS_PT)])
        plsc.subcore_barrier()

        pltpu.async_copy(yc.at[src_v.at[0, 0]], rows0, sg0)
        pltpu.async_copy(yc.at[src_v.at[0, 1]], rows1, sg1)
        pltpu.async_copy(src2_hbm.at[s, 1], src_v.at[1], si)
        pltpu.async_copy(dst_hbm.at[s, 1], dst_v.at[1], si)

        def outer(g, _):
            p = lax.rem(g, 2)

            def body(i, _):
                a = 2 * i
                b = a + 1
                pltpu.make_async_copy(
                    yc.at[src_v.at[p, a]], rows0, sg0).wait()
                pltpu.async_copy(rows0, acc_sh.at[dst_v.at[p, a]], ss0,
                                 add=True)
                pltpu.make_async_copy(
                    rows0, acc_sh.at[dst_v.at[p, a]], ss0).wait()

                @pl.when(a + 2 < GC)
                def _():
                    pltpu.async_copy(yc.at[src_v.at[p, a + 2]], rows0,
                                     sg0)

                pltpu.make_async_copy(
                    yc.at[src_v.at[p, b]], rows1, sg1).wait()
                pltpu.async_copy(rows1, acc_sh.at[dst_v.at[p, b]], ss1,
                                 add=True)
                pltpu.make_async_copy(
                    rows1, acc_sh.at[dst_v.at[p, b]], ss1).wait()

                @pl.when(b + 2 < GC)
                def _():
                    pltpu.async_copy(yc.at[src_v.at[p, b + 2]], rows1,
                                     sg1)
                return 0
            lax.fori_loop(0, GC // 2, body, 0)

            @pl.when(g + 1 < G)
            def _():
                q = 1 - p
                pltpu.make_async_copy(
                    src2_hbm.at[s, g + 1], src_v.at[q], si).wait()
                pltpu.make_async_copy(
                    dst_hbm.at[s, g + 1], dst_v.at[q], si).wait()
                pltpu.async_copy(yc.at[src_v.at[q, 0]], rows0, sg0)
                pltpu.async_copy(yc.at[src_v.at[q, 1]], rows1, sg1)

                @pl.when(g + 2 < G)
                def _():
                    pltpu.async_copy(src2_hbm.at[s, g + 2], src_v.at[p],
                                     si)
                    pltpu.async_copy(dst_hbm.at[s, g + 2], dst_v.at[p], si)
            return 0
        lax.fori_loop(0, G, outer, 0)

        plsc.subcore_barrier()
        pltpu.sync_copy(acc_sh.at[pl.ds(s * ROWS_PT, ROWS_PT)],
                        z_hbm.at[c, pl.ds(s * ROWS_PT, ROWS_PT)])

    return agg


_agg128 = _make_agg(128)
_agg32 = _make_agg(32)


def _tc_first(xp, W1, degA, degB):
    def body(x_ref, w_ref, da_ref, db_ref, y_ref, dv_ref):
        dv = lax.rsqrt(1.0 + da_ref[...] + db_ref[...])
        xw = jnp.dot(x_ref[...], w_ref[...],
                     preferred_element_type=jnp.float32)
        y = (xw * dv[:, None]).astype(jnp.bfloat16)
        y_ref[0] = y[:, :128]
        y_ref[1] = y[:, 128:]
        dv_ref[...] = dv

    return pl.pallas_call(
        body,
        grid=(GR,),
        in_specs=[
            pl.BlockSpec((BR, D_IN), lambda i: (i, 0)),
            pl.BlockSpec((D_IN, D_H), lambda i: (0, 0)),
            pl.BlockSpec((BR,), lambda i: (i,)),
            pl.BlockSpec((BR,), lambda i: (i,)),
        ],
        out_specs=[
            pl.BlockSpec((2, BR, 128), lambda i: (0, i, 0)),
            pl.BlockSpec((BR,), lambda i: (i,)),
        ],
        out_shape=[
            jax.ShapeDtypeStruct((2, NP, 128), jnp.bfloat16),
            jax.ShapeDtypeStruct((NP,), jnp.float32),
        ],
    )(xp, W1, degA, degB)


def _tc_mid(z, dinv, W, b, g, beta, d_out):
    dh = d_out // 2

    def body(z_ref, dv_ref, w_ref, b_ref, g_ref, bt_ref, y_ref):
        dv = dv_ref[...]
        zf = jnp.concatenate([z_ref[0], z_ref[1]],
                             axis=1).astype(jnp.float32)
        sg = g_ref[...] * _BN_S
        t = b_ref[...] * sg + bt_ref[...]
        h = jnp.maximum(zf * dv[:, None] * sg + t, 0.0)
        xw = jnp.dot(h, w_ref[...], preferred_element_type=jnp.float32)
        y = (xw * dv[:, None]).astype(jnp.bfloat16)
        y_ref[0] = y[:, :dh]
        y_ref[1] = y[:, dh:]

    return pl.pallas_call(
        body,
        grid=(GR,),
        in_specs=[
            pl.BlockSpec((2, BR, 128), lambda i: (0, i, 0)),
            pl.BlockSpec((BR,), lambda i: (i,)),
            pl.BlockSpec((D_H, d_out), lambda i: (0, 0)),
            pl.BlockSpec((1, D_H), lambda i: (0, 0)),
            pl.BlockSpec((1, D_H), lambda i: (0, 0)),
            pl.BlockSpec((1, D_H), lambda i: (0, 0)),
        ],
        out_specs=pl.BlockSpec((2, BR, dh), lambda i: (0, i, 0)),
        out_shape=jax.ShapeDtypeStruct((2, NP, dh), jnp.bfloat16),
    )(z, dinv, W, b.reshape(1, -1), g.reshape(1, -1), beta.reshape(1, -1))


def _tc_softmax(z3, dinv, b3):
    def body(z_ref, dv_ref, b_ref, o_ref):
        zf = jnp.concatenate([z_ref[0], z_ref[1]],
                             axis=1).astype(jnp.float32)
        u = zf * dv_ref[...][:, None] + b_ref[...]
        m = jnp.max(u, axis=1, keepdims=True)
        e = jnp.exp(u - m)
        o_ref[...] = (u - m) - jnp.log(jnp.sum(e, axis=1, keepdims=True))

    return pl.pallas_call(
        body,
        grid=(GR,),
        in_specs=[
            pl.BlockSpec((2, BR, 32), lambda i: (0, i, 0)),
            pl.BlockSpec((BR,), lambda i: (i,)),
            pl.BlockSpec((1, D_OUT), lambda i: (0, 0)),
        ],
        out_specs=pl.BlockSpec((BR, D_OUT), lambda i: (i, 0)),
        out_shape=jax.ShapeDtypeStruct((NP, D_OUT), jnp.float32),
    )(z3, dinv, b3.reshape(1, -1))


def kernel(x, adj_t, W1, b1, g1, beta1, W2, b2, g2, beta2, W3, b3):
    src = adj_t[0]
    dst = adj_t[1]
    npad = EPAD - E
    padrows = N + (jnp.arange(npad, dtype=jnp.int32) % (NP - N))
    srcp = jnp.concatenate([src, padrows])
    dstp = jnp.concatenate([dst, padrows])
    src2 = srcp.reshape(NTILE, G, GC, CH)
    dstr = dstp.reshape(NTILE, G, GC, CH)
    degr = dstp.reshape(NCORE, NTILE, CW, CHD)
    xp = jnp.pad(x, ((0, NP - N), (0, 0)))

    deg = _deg_kernel(degr)
    y1, dinv = _tc_first(xp, W1, deg[0], deg[1])
    z1 = _agg128(y1, src2, dstr)
    y2 = _tc_mid(z1, dinv, W2, b1, g1, beta1, D_H)
    z2 = _agg128(y2, src2, dstr)
    y3 = _tc_mid(z2, dinv, W3, b2, g2, beta2, D_OUT)
    z3 = _agg32(y3, src2, dstr)
    out = _tc_softmax(z3, dinv, b3)
    return out[:N]


# speedup vs baseline: 1.1202x; 1.1202x over previous
import functools

import numpy as np
import jax
import jax.numpy as jnp
from jax import lax
from jax.experimental import pallas as pl
from jax.experimental.pallas import tpu as pltpu
from jax.experimental.pallas import tpu_sc as plsc

N = 10000
NP = 10240
E = 320000
EPAD = 327680
D_IN = 128
D_H = 256
D_OUT = 64

NCORE = 2
NTILE = 16
ROWS_PT = NP // NTILE

CHD = 128
EW = EPAD // (NCORE * NTILE)
CW = EW // CHD

CH = 128
EPT = EPAD // NTILE
CT = EPT // CH
GC = 16
G = CT // GC

BR = 1024
GR = NP // BR

_BN_S = float(1.0 / np.sqrt(1.0 + 1e-5))

_MESH = plsc.VectorSubcoreMesh(
    core_axis_name="c", subcore_axis_name="s", num_cores=NCORE,
    num_subcores=NTILE)


@functools.partial(
    pl.kernel,
    out_type=jax.ShapeDtypeStruct((NCORE, NP), jnp.float32),
    mesh=_MESH,
    scratch_types=[
        pltpu.VMEM((CW, CHD), jnp.int32),
        pltpu.VMEM((CHD,), jnp.float32),
        pltpu.VMEM((ROWS_PT,), jnp.float32),
        pltpu.VMEM_SHARED((NP,), jnp.float32),
        pltpu.SemaphoreType.DMA,
    ],
)
def _deg_kernel(dst_hbm, deg_out, dst_v, ones_v, zb_v, acc_sh, sem):
    c = lax.axis_index("c")
    s = lax.axis_index("s")

    def fz(i, _):
        zb_v[pl.ds(i * 16, 16)] = jnp.zeros((16,), jnp.float32)
        return 0
    lax.fori_loop(0, ROWS_PT // 16, fz, 0)

    def fo(i, _):
        ones_v[pl.ds(i * 16, 16)] = jnp.ones((16,), jnp.float32)
        return 0
    lax.fori_loop(0, CHD // 16, fo, 0)

    pltpu.sync_copy(dst_hbm.at[c, s], dst_v)
    pltpu.sync_copy(zb_v, acc_sh.at[pl.ds(s * ROWS_PT, ROWS_PT)])
    plsc.subcore_barrier()

    def body(i, _):
        pltpu.async_copy(ones_v, acc_sh.at[dst_v.at[i]], sem, add=True)

        @pl.when(i >= 8)
        def _():
            pltpu.make_async_copy(ones_v, acc_sh.at[dst_v.at[i]], sem).wait()
        return 0
    lax.fori_loop(0, CW, body, 0)

    def drain(i, _):
        pltpu.make_async_copy(ones_v, acc_sh.at[dst_v.at[0]], sem).wait()
        return 0
    lax.fori_loop(0, 8, drain, 0)

    plsc.subcore_barrier()
    pltpu.sync_copy(acc_sh.at[pl.ds(s * ROWS_PT, ROWS_PT)],
                    deg_out.at[c, pl.ds(s * ROWS_PT, ROWS_PT)])


def _make_agg(d_half):
    @functools.partial(
        pl.kernel,
        out_type=jax.ShapeDtypeStruct((NCORE, NP, d_half), jnp.bfloat16),
        mesh=_MESH,
        compiler_params=pltpu.CompilerParams(use_tc_tiling_on_sc=False),
        scratch_types=[
            pltpu.VMEM((2, GC, CH), jnp.int32),
            pltpu.VMEM((2, GC, CH), jnp.int32),
            pltpu.VMEM((CH, d_half), jnp.bfloat16),
            pltpu.VMEM((CH, d_half), jnp.bfloat16),
            pltpu.VMEM((CH, d_half), jnp.bfloat16),
            pltpu.VMEM((CH, d_half), jnp.bfloat16),
            pltpu.VMEM_SHARED((NP, d_half), jnp.bfloat16),
            pltpu.SemaphoreType.DMA,
            pltpu.SemaphoreType.DMA,
            pltpu.SemaphoreType.DMA,
            pltpu.SemaphoreType.DMA,
            pltpu.SemaphoreType.DMA,
            pltpu.SemaphoreType.DMA,
            pltpu.SemaphoreType.DMA,
            pltpu.SemaphoreType.DMA,
            pltpu.SemaphoreType.DMA,
        ],
    )
    def agg(y_hbm, src2_hbm, dst_hbm, z_hbm, src_v, dst_v, r0, r1, r2, r3,
            acc_sh, si, sg0, sg1, sg2, sg3, ss0, ss1, ss2, ss3):
        c = lax.axis_index("c")
        s = lax.axis_index("s")
        yc = y_hbm.at[c]
        rows = (r0, r1, r2, r3)
        sg = (sg0, sg1, sg2, sg3)
        ss = (ss0, ss1, ss2, ss3)
        pltpu.sync_copy(src2_hbm.at[s, 0], src_v.at[0])
        pltpu.sync_copy(dst_hbm.at[s, 0], dst_v.at[0])
        pltpu.sync_copy(yc.at[pl.ds(s * ROWS_PT, ROWS_PT)],
                        acc_sh.at[pl.ds(s * ROWS_PT, ROWS_PT)])
        plsc.subcore_barrier()

        for k in range(4):
            pltpu.async_copy(yc.at[src_v.at[0, k]], rows[k], sg[k])
        pltpu.async_copy(src2_hbm.at[s, 1], src_v.at[1], si)
        pltpu.async_copy(dst_hbm.at[s, 1], dst_v.at[1], si)

        def outer(g, _):
            p = lax.rem(g, 2)

            def body(i, _):
                for k in range(4):
                    ch = 4 * i + k
                    pltpu.make_async_copy(
                        yc.at[src_v.at[p, ch]], rows[k], sg[k]).wait()
                    pltpu.async_copy(rows[k], acc_sh.at[dst_v.at[p, ch]],
                                     ss[k], add=True)
                for k in range(4):
                    ch = 4 * i + k
                    pltpu.make_async_copy(
                        rows[k], acc_sh.at[dst_v.at[p, ch]], ss[k]).wait()

                    @pl.when(ch + 4 < GC)
                    def _(k=k, ch=ch):
                        pltpu.async_copy(yc.at[src_v.at[p, ch + 4]],
                                         rows[k], sg[k])
                return 0
            lax.fori_loop(0, GC // 4, body, 0)

            @pl.when(g + 1 < G)
            def _():
                q = 1 - p
                pltpu.make_async_copy(
                    src2_hbm.at[s, g + 1], src_v.at[q], si).wait()
                pltpu.make_async_copy(
                    dst_hbm.at[s, g + 1], dst_v.at[q], si).wait()
                for k in range(4):
                    pltpu.async_copy(yc.at[src_v.at[q, k]], rows[k], sg[k])

                @pl.when(g + 2 < G)
                def _():
                    pltpu.async_copy(src2_hbm.at[s, g + 2], src_v.at[p],
                                     si)
                    pltpu.async_copy(dst_hbm.at[s, g + 2], dst_v.at[p], si)
            return 0
        lax.fori_loop(0, G, outer, 0)

        plsc.subcore_barrier()
        pltpu.sync_copy(acc_sh.at[pl.ds(s * ROWS_PT, ROWS_PT)],
                        z_hbm.at[c, pl.ds(s * ROWS_PT, ROWS_PT)])

    return agg


_agg128 = _make_agg(128)
_agg32 = _make_agg(32)


def _tc_first(xp, W1, degA, degB):
    def body(x_ref, w_ref, da_ref, db_ref, y_ref, dv_ref):
        dv = lax.rsqrt(1.0 + da_ref[...] + db_ref[...])
        xw = jnp.dot(x_ref[...], w_ref[...],
                     preferred_element_type=jnp.float32)
        y = (xw * dv[:, None]).astype(jnp.bfloat16)
        y_ref[0] = y[:, :128]
        y_ref[1] = y[:, 128:]
        dv_ref[...] = dv

    return pl.pallas_call(
        body,
        grid=(GR,),
        in_specs=[
            pl.BlockSpec((BR, D_IN), lambda i: (i, 0)),
            pl.BlockSpec((D_IN, D_H), lambda i: (0, 0)),
            pl.BlockSpec((BR,), lambda i: (i,)),
            pl.BlockSpec((BR,), lambda i: (i,)),
        ],
        out_specs=[
            pl.BlockSpec((2, BR, 128), lambda i: (0, i, 0)),
            pl.BlockSpec((BR,), lambda i: (i,)),
        ],
        out_shape=[
            jax.ShapeDtypeStruct((2, NP, 128), jnp.bfloat16),
            jax.ShapeDtypeStruct((NP,), jnp.float32),
        ],
    )(xp, W1, degA, degB)


def _tc_mid(z, dinv, W, b, g, beta, d_out):
    dh = d_out // 2

    def body(z_ref, dv_ref, w_ref, b_ref, g_ref, bt_ref, y_ref):
        dv = dv_ref[...]
        zf = jnp.concatenate([z_ref[0], z_ref[1]],
                             axis=1).astype(jnp.float32)
        sg = g_ref[...] * _BN_S
        t = b_ref[...] * sg + bt_ref[...]
        h = jnp.maximum(zf * dv[:, None] * sg + t, 0.0)
        xw = jnp.dot(h, w_ref[...], preferred_element_type=jnp.float32)
        y = (xw * dv[:, None]).astype(jnp.bfloat16)
        y_ref[0] = y[:, :dh]
        y_ref[1] = y[:, dh:]

    return pl.pallas_call(
        body,
        grid=(GR,),
        in_specs=[
            pl.BlockSpec((2, BR, 128), lambda i: (0, i, 0)),
            pl.BlockSpec((BR,), lambda i: (i,)),
            pl.BlockSpec((D_H, d_out), lambda i: (0, 0)),
            pl.BlockSpec((1, D_H), lambda i: (0, 0)),
            pl.BlockSpec((1, D_H), lambda i: (0, 0)),
            pl.BlockSpec((1, D_H), lambda i: (0, 0)),
        ],
        out_specs=pl.BlockSpec((2, BR, dh), lambda i: (0, i, 0)),
        out_shape=jax.ShapeDtypeStruct((2, NP, dh), jnp.bfloat16),
    )(z, dinv, W, b.reshape(1, -1), g.reshape(1, -1), beta.reshape(1, -1))


def _tc_softmax(z3, dinv, b3):
    def body(z_ref, dv_ref, b_ref, o_ref):
        zf = jnp.concatenate([z_ref[0], z_ref[1]],
                             axis=1).astype(jnp.float32)
        u = zf * dv_ref[...][:, None] + b_ref[...]
        m = jnp.max(u, axis=1, keepdims=True)
        e = jnp.exp(u - m)
        o_ref[...] = (u - m) - jnp.log(jnp.sum(e, axis=1, keepdims=True))

    return pl.pallas_call(
        body,
        grid=(GR,),
        in_specs=[
            pl.BlockSpec((2, BR, 32), lambda i: (0, i, 0)),
            pl.BlockSpec((BR,), lambda i: (i,)),
            pl.BlockSpec((1, D_OUT), lambda i: (0, 0)),
        ],
        out_specs=pl.BlockSpec((BR, D_OUT), lambda i: (i, 0)),
        out_shape=jax.ShapeDtypeStruct((NP, D_OUT), jnp.float32),
    )(z3, dinv, b3.reshape(1, -1))


def kernel(x, adj_t, W1, b1, g1, beta1, W2, b2, g2, beta2, W3, b3):
    src = adj_t[0]
    dst = adj_t[1]
    npad = EPAD - E
    padrows = N + (jnp.arange(npad, dtype=jnp.int32) % (NP - N))
    srcp = jnp.concatenate([src, padrows])
    dstp = jnp.concatenate([dst, padrows])
    src2 = srcp.reshape(NTILE, G, GC, CH)
    dstr = dstp.reshape(NTILE, G, GC, CH)
    degr = dstp.reshape(NCORE, NTILE, CW, CHD)
    xp = jnp.pad(x, ((0, NP - N), (0, 0)))

    deg = _deg_kernel(degr)
    y1, dinv = _tc_first(xp, W1, deg[0], deg[1])
    z1 = _agg128(y1, src2, dstr)
    y2 = _tc_mid(z1, dinv, W2, b1, g1, beta1, D_H)
    z2 = _agg128(y2, src2, dstr)
    y3 = _tc_mid(z2, dinv, W3, b2, g2, beta2, D_OUT)
    z3 = _agg32(y3, src2, dstr)
    out = _tc_softmax(z3, dinv, b3)
    return out[:N]


# speedup vs baseline: 1.1412x; 1.0187x over previous
import functools

import numpy as np
import jax
import jax.numpy as jnp
from jax import lax
from jax.experimental import pallas as pl
from jax.experimental.pallas import tpu as pltpu
from jax.experimental.pallas import tpu_sc as plsc

N = 10000
NP = 10240
E = 320000
EPAD = 327680
D_IN = 128
D_H = 256
D_OUT = 64

NCORE = 2
NTILE = 16
ROWS_PT = NP // NTILE

CHD = 128
EW = EPAD // (NCORE * NTILE)
CW = EW // CHD

CH = 128
EPT = EPAD // NTILE
CT = EPT // CH
GC = 32
G = CT // GC

BR = 1024
GR = NP // BR

_BN_S = float(1.0 / np.sqrt(1.0 + 1e-5))

_MESH = plsc.VectorSubcoreMesh(
    core_axis_name="c", subcore_axis_name="s", num_cores=NCORE,
    num_subcores=NTILE)


@functools.partial(
    pl.kernel,
    out_type=jax.ShapeDtypeStruct((NCORE, NP), jnp.float32),
    mesh=_MESH,
    scratch_types=[
        pltpu.VMEM((CW, CHD), jnp.int32),
        pltpu.VMEM((CHD,), jnp.float32),
        pltpu.VMEM((ROWS_PT,), jnp.float32),
        pltpu.VMEM_SHARED((NP,), jnp.float32),
        pltpu.SemaphoreType.DMA,
    ],
)
def _deg_kernel(dst_hbm, deg_out, dst_v, ones_v, zb_v, acc_sh, sem):
    c = lax.axis_index("c")
    s = lax.axis_index("s")

    def fz(i, _):
        zb_v[pl.ds(i * 16, 16)] = jnp.zeros((16,), jnp.float32)
        return 0
    lax.fori_loop(0, ROWS_PT // 16, fz, 0)

    def fo(i, _):
        ones_v[pl.ds(i * 16, 16)] = jnp.ones((16,), jnp.float32)
        return 0
    lax.fori_loop(0, CHD // 16, fo, 0)

    pltpu.sync_copy(dst_hbm.at[c, s], dst_v)
    pltpu.sync_copy(zb_v, acc_sh.at[pl.ds(s * ROWS_PT, ROWS_PT)])
    plsc.subcore_barrier()

    def body(i, _):
        pltpu.async_copy(ones_v, acc_sh.at[dst_v.at[i]], sem, add=True)

        @pl.when(i >= 8)
        def _():
            pltpu.make_async_copy(ones_v, acc_sh.at[dst_v.at[i]], sem).wait()
        return 0
    lax.fori_loop(0, CW, body, 0)

    def drain(i, _):
        pltpu.make_async_copy(ones_v, acc_sh.at[dst_v.at[0]], sem).wait()
        return 0
    lax.fori_loop(0, 8, drain, 0)

    plsc.subcore_barrier()
    pltpu.sync_copy(acc_sh.at[pl.ds(s * ROWS_PT, ROWS_PT)],
                    deg_out.at[c, pl.ds(s * ROWS_PT, ROWS_PT)])


def _make_agg(d_half):
    @functools.partial(
        pl.kernel,
        out_type=jax.ShapeDtypeStruct((NCORE, NP, d_half), jnp.bfloat16),
        mesh=_MESH,
        compiler_params=pltpu.CompilerParams(use_tc_tiling_on_sc=False),
        scratch_types=[
            pltpu.VMEM((2, GC, CH), jnp.int32),
            pltpu.VMEM((2, GC, CH), jnp.int32),
            pltpu.VMEM((CH, d_half), jnp.bfloat16),
            pltpu.VMEM((CH, d_half), jnp.bfloat16),
            pltpu.VMEM((CH, d_half), jnp.bfloat16),
            pltpu.VMEM((CH, d_half), jnp.bfloat16),
            pltpu.VMEM_SHARED((NP, d_half), jnp.bfloat16),
            pltpu.SemaphoreType.DMA,
            pltpu.SemaphoreType.DMA,
            pltpu.SemaphoreType.DMA,
            pltpu.SemaphoreType.DMA,
            pltpu.SemaphoreType.DMA,
            pltpu.SemaphoreType.DMA,
            pltpu.SemaphoreType.DMA,
            pltpu.SemaphoreType.DMA,
            pltpu.SemaphoreType.DMA,
        ],
    )
    def agg(y_hbm, src2_hbm, dst_hbm, z_hbm, src_v, dst_v, r0, r1, r2, r3,
            acc_sh, si, sg0, sg1, sg2, sg3, ss0, ss1, ss2, ss3):
        c = lax.axis_index("c")
        s = lax.axis_index("s")
        yc = y_hbm.at[c]
        rows = (r0, r1, r2, r3)
        sg = (sg0, sg1, sg2, sg3)
        ss = (ss0, ss1, ss2, ss3)
        pltpu.sync_copy(src2_hbm.at[s, 0], src_v.at[0])
        pltpu.sync_copy(dst_hbm.at[s, 0], dst_v.at[0])
        pltpu.sync_copy(yc.at[pl.ds(s * ROWS_PT, ROWS_PT)],
                        acc_sh.at[pl.ds(s * ROWS_PT, ROWS_PT)])
        plsc.subcore_barrier()

        for k in range(4):
            pltpu.async_copy(yc.at[src_v.at[0, k]], rows[k], sg[k])
        pltpu.async_copy(src2_hbm.at[s, 1], src_v.at[1], si)
        pltpu.async_copy(dst_hbm.at[s, 1], dst_v.at[1], si)

        def outer(g, _):
            p = lax.rem(g, 2)

            def body(i, _):
                for k in range(4):
                    ch = 4 * i + k
                    pltpu.make_async_copy(
                        yc.at[src_v.at[p, ch]], rows[k], sg[k]).wait()
                    pltpu.async_copy(rows[k], acc_sh.at[dst_v.at[p, ch]],
                                     ss[k], add=True)
                for k in range(4):
                    ch = 4 * i + k
                    pltpu.make_async_copy(
                        rows[k], acc_sh.at[dst_v.at[p, ch]], ss[k]).wait()

                    @pl.when(ch + 4 < GC)
                    def _(k=k, ch=ch):
                        pltpu.async_copy(yc.at[src_v.at[p, ch + 4]],
                                         rows[k], sg[k])
                return 0
            lax.fori_loop(0, GC // 4, body, 0)

            @pl.when(g + 1 < G)
            def _():
                q = 1 - p
                pltpu.make_async_copy(
                    src2_hbm.at[s, g + 1], src_v.at[q], si).wait()
                pltpu.make_async_copy(
                    dst_hbm.at[s, g + 1], dst_v.at[q], si).wait()
                for k in range(4):
                    pltpu.async_copy(yc.at[src_v.at[q, k]], rows[k], sg[k])

                @pl.when(g + 2 < G)
                def _():
                    pltpu.async_copy(src2_hbm.at[s, g + 2], src_v.at[p],
                                     si)
                    pltpu.async_copy(dst_hbm.at[s, g + 2], dst_v.at[p], si)
            return 0
        lax.fori_loop(0, G, outer, 0)

        plsc.subcore_barrier()
        pltpu.sync_copy(acc_sh.at[pl.ds(s * ROWS_PT, ROWS_PT)],
                        z_hbm.at[c, pl.ds(s * ROWS_PT, ROWS_PT)])

    return agg


_agg128 = _make_agg(128)
_agg32 = _make_agg(32)


def _tc_first(xp, W1, degA, degB):
    def body(x_ref, w_ref, da_ref, db_ref, y_ref, dv_ref):
        dv = lax.rsqrt(1.0 + da_ref[...] + db_ref[...])
        xw = jnp.dot(x_ref[...], w_ref[...],
                     preferred_element_type=jnp.float32)
        y = (xw * dv[:, None]).astype(jnp.bfloat16)
        y_ref[0] = y[:, :128]
        y_ref[1] = y[:, 128:]
        dv_ref[...] = dv

    return pl.pallas_call(
        body,
        grid=(GR,),
        in_specs=[
            pl.BlockSpec((BR, D_IN), lambda i: (i, 0)),
            pl.BlockSpec((D_IN, D_H), lambda i: (0, 0)),
            pl.BlockSpec((BR,), lambda i: (i,)),
            pl.BlockSpec((BR,), lambda i: (i,)),
        ],
        out_specs=[
            pl.BlockSpec((2, BR, 128), lambda i: (0, i, 0)),
            pl.BlockSpec((BR,), lambda i: (i,)),
        ],
        out_shape=[
            jax.ShapeDtypeStruct((2, NP, 128), jnp.bfloat16),
            jax.ShapeDtypeStruct((NP,), jnp.float32),
        ],
    )(xp, W1, degA, degB)


def _tc_mid(z, dinv, W, b, g, beta, d_out):
    dh = d_out // 2

    def body(z_ref, dv_ref, w_ref, b_ref, g_ref, bt_ref, y_ref):
        dv = dv_ref[...]
        zf = jnp.concatenate([z_ref[0], z_ref[1]],
                             axis=1).astype(jnp.float32)
        sg = g_ref[...] * _BN_S
        t = b_ref[...] * sg + bt_ref[...]
        h = jnp.maximum(zf * dv[:, None] * sg + t, 0.0)
        xw = jnp.dot(h, w_ref[...], preferred_element_type=jnp.float32)
        y = (xw * dv[:, None]).astype(jnp.bfloat16)
        y_ref[0] = y[:, :dh]
        y_ref[1] = y[:, dh:]

    return pl.pallas_call(
        body,
        grid=(GR,),
        in_specs=[
            pl.BlockSpec((2, BR, 128), lambda i: (0, i, 0)),
            pl.BlockSpec((BR,), lambda i: (i,)),
            pl.BlockSpec((D_H, d_out), lambda i: (0, 0)),
            pl.BlockSpec((1, D_H), lambda i: (0, 0)),
            pl.BlockSpec((1, D_H), lambda i: (0, 0)),
            pl.BlockSpec((1, D_H), lambda i: (0, 0)),
        ],
        out_specs=pl.BlockSpec((2, BR, dh), lambda i: (0, i, 0)),
        out_shape=jax.ShapeDtypeStruct((2, NP, dh), jnp.bfloat16),
    )(z, dinv, W, b.reshape(1, -1), g.reshape(1, -1), beta.reshape(1, -1))


def _tc_softmax(z3, dinv, b3):
    def body(z_ref, dv_ref, b_ref, o_ref):
        zf = jnp.concatenate([z_ref[0], z_ref[1]],
                             axis=1).astype(jnp.float32)
        u = zf * dv_ref[...][:, None] + b_ref[...]
        m = jnp.max(u, axis=1, keepdims=True)
        e = jnp.exp(u - m)
        o_ref[...] = (u - m) - jnp.log(jnp.sum(e, axis=1, keepdims=True))

    return pl.pallas_call(
        body,
        grid=(GR,),
        in_specs=[
            pl.BlockSpec((2, BR, 32), lambda i: (0, i, 0)),
            pl.BlockSpec((BR,), lambda i: (i,)),
            pl.BlockSpec((1, D_OUT), lambda i: (0, 0)),
        ],
        out_specs=pl.BlockSpec((BR, D_OUT), lambda i: (i, 0)),
        out_shape=jax.ShapeDtypeStruct((NP, D_OUT), jnp.float32),
    )(z3, dinv, b3.reshape(1, -1))


def kernel(x, adj_t, W1, b1, g1, beta1, W2, b2, g2, beta2, W3, b3):
    src = adj_t[0]
    dst = adj_t[1]
    npad = EPAD - E
    padrows = N + (jnp.arange(npad, dtype=jnp.int32) % (NP - N))
    srcp = jnp.concatenate([src, padrows])
    dstp = jnp.concatenate([dst, padrows])
    src2 = srcp.reshape(NTILE, G, GC, CH)
    dstr = dstp.reshape(NTILE, G, GC, CH)
    degr = dstp.reshape(NCORE, NTILE, CW, CHD)
    xp = jnp.pad(x, ((0, NP - N), (0, 0)))

    deg = _deg_kernel(degr)
    y1, dinv = _tc_first(xp, W1, deg[0], deg[1])
    z1 = _agg128(y1, src2, dstr)
    y2 = _tc_mid(z1, dinv, W2, b1, g1, beta1, D_H)
    z2 = _agg128(y2, src2, dstr)
    y3 = _tc_mid(z2, dinv, W3, b2, g2, beta2, D_OUT)
    z3 = _agg32(y3, src2, dstr)
    out = _tc_softmax(z3, dinv, b3)
    return out[:N]


# speedup vs baseline: 1.1956x; 1.0477x over previous
import functools

import numpy as np
import jax
import jax.numpy as jnp
from jax import lax
from jax.experimental import pallas as pl
from jax.experimental.pallas import tpu as pltpu
from jax.experimental.pallas import tpu_sc as plsc

N = 10000
NP = 10240
E = 320000
EPAD = 327680
D_IN = 128
D_H = 256
D_OUT = 64

NCORE = 2
NTILE = 16
ROWS_PT = NP // NTILE

CHD = 128
EW = EPAD // (NCORE * NTILE)
CW = EW // CHD

CH = 128
EPT = EPAD // NTILE
CT = EPT // CH
GC = 32
G = CT // GC

BR = 1024
GR = NP // BR

_BN_S = float(1.0 / np.sqrt(1.0 + 1e-5))

_MESH = plsc.VectorSubcoreMesh(
    core_axis_name="c", subcore_axis_name="s", num_cores=NCORE,
    num_subcores=NTILE)


@functools.partial(
    pl.kernel,
    out_type=jax.ShapeDtypeStruct((NCORE, NP), jnp.float32),
    mesh=_MESH,
    scratch_types=[
        pltpu.VMEM((CW, CHD), jnp.int32),
        pltpu.VMEM((CHD,), jnp.float32),
        pltpu.VMEM((ROWS_PT,), jnp.float32),
        pltpu.VMEM_SHARED((NP,), jnp.float32),
        pltpu.SemaphoreType.DMA,
    ],
)
def _deg_kernel(dst_hbm, deg_out, dst_v, ones_v, zb_v, acc_sh, sem):
    c = lax.axis_index("c")
    s = lax.axis_index("s")

    def fz(i, _):
        zb_v[pl.ds(i * 16, 16)] = jnp.zeros((16,), jnp.float32)
        return 0
    lax.fori_loop(0, ROWS_PT // 16, fz, 0)

    def fo(i, _):
        ones_v[pl.ds(i * 16, 16)] = jnp.ones((16,), jnp.float32)
        return 0
    lax.fori_loop(0, CHD // 16, fo, 0)

    pltpu.sync_copy(dst_hbm.at[c, s], dst_v)
    pltpu.sync_copy(zb_v, acc_sh.at[pl.ds(s * ROWS_PT, ROWS_PT)])
    plsc.subcore_barrier()

    def body(i, _):
        pltpu.async_copy(ones_v, acc_sh.at[dst_v.at[i]], sem, add=True)

        @pl.when(i >= 8)
        def _():
            pltpu.make_async_copy(ones_v, acc_sh.at[dst_v.at[i]], sem).wait()
        return 0
    lax.fori_loop(0, CW, body, 0)

    def drain(i, _):
        pltpu.make_async_copy(ones_v, acc_sh.at[dst_v.at[0]], sem).wait()
        return 0
    lax.fori_loop(0, 8, drain, 0)

    plsc.subcore_barrier()
    pltpu.sync_copy(acc_sh.at[pl.ds(s * ROWS_PT, ROWS_PT)],
                    deg_out.at[c, pl.ds(s * ROWS_PT, ROWS_PT)])


def _make_agg(d_half):
    @functools.partial(
        pl.kernel,
        out_type=jax.ShapeDtypeStruct((NCORE, NP, d_half), jnp.bfloat16),
        mesh=_MESH,
        compiler_params=pltpu.CompilerParams(use_tc_tiling_on_sc=False),
        scratch_types=[
            pltpu.VMEM((2, GC, CH), jnp.int32),
            pltpu.VMEM((2, GC, CH), jnp.int32),
            pltpu.VMEM((CH, d_half), jnp.bfloat16),
            pltpu.VMEM((CH, d_half), jnp.bfloat16),
            pltpu.VMEM((CH, d_half), jnp.bfloat16),
            pltpu.VMEM((CH, d_half), jnp.bfloat16),
            pltpu.VMEM((CH, d_half), jnp.bfloat16),
            pltpu.VMEM((CH, d_half), jnp.bfloat16),
            pltpu.VMEM((CH, d_half), jnp.bfloat16),
            pltpu.VMEM((CH, d_half), jnp.bfloat16),
            pltpu.VMEM_SHARED((NP, d_half), jnp.bfloat16),
        ] + [pltpu.SemaphoreType.DMA] * 17,
    )
    def agg(y_hbm, src2_hbm, dst_hbm, z_hbm, src_v, dst_v, r0, r1, r2, r3,
            r4, r5, r6, r7, acc_sh, si, sg0, sg1, sg2, sg3, sg4, sg5, sg6,
            sg7, ss0, ss1, ss2, ss3, ss4, ss5, ss6, ss7):
        c = lax.axis_index("c")
        s = lax.axis_index("s")
        yc = y_hbm.at[c]
        rows = (r0, r1, r2, r3, r4, r5, r6, r7)
        sg = (sg0, sg1, sg2, sg3, sg4, sg5, sg6, sg7)
        ss = (ss0, ss1, ss2, ss3, ss4, ss5, ss6, ss7)
        pltpu.sync_copy(src2_hbm.at[s, 0], src_v.at[0])
        pltpu.sync_copy(dst_hbm.at[s, 0], dst_v.at[0])
        pltpu.sync_copy(yc.at[pl.ds(s * ROWS_PT, ROWS_PT)],
                        acc_sh.at[pl.ds(s * ROWS_PT, ROWS_PT)])
        plsc.subcore_barrier()

        for k in range(8):
            pltpu.async_copy(yc.at[src_v.at[0, k]], rows[k], sg[k])
        pltpu.async_copy(src2_hbm.at[s, 1], src_v.at[1], si)
        pltpu.async_copy(dst_hbm.at[s, 1], dst_v.at[1], si)

        def outer(g, _):
            p = lax.rem(g, 2)

            def body(i, _):
                for k in range(8):
                    ch = 8 * i + k
                    pltpu.make_async_copy(
                        yc.at[src_v.at[p, ch]], rows[k], sg[k]).wait()
                    pltpu.async_copy(rows[k], acc_sh.at[dst_v.at[p, ch]],
                                     ss[k], add=True)
                for k in range(8):
                    ch = 8 * i + k
                    pltpu.make_async_copy(
                        rows[k], acc_sh.at[dst_v.at[p, ch]], ss[k]).wait()

                    @pl.when(ch + 8 < GC)
                    def _(k=k, ch=ch):
                        pltpu.async_copy(yc.at[src_v.at[p, ch + 8]],
                                         rows[k], sg[k])
                return 0
            lax.fori_loop(0, GC // 8, body, 0)

            @pl.when(g + 1 < G)
            def _():
                q = 1 - p
                pltpu.make_async_copy(
                    src2_hbm.at[s, g + 1], src_v.at[q], si).wait()
                pltpu.make_async_copy(
                    dst_hbm.at[s, g + 1], dst_v.at[q], si).wait()
                for k in range(8):
                    pltpu.async_copy(yc.at[src_v.at[q, k]], rows[k], sg[k])

                @pl.when(g + 2 < G)
                def _():
                    pltpu.async_copy(src2_hbm.at[s, g + 2], src_v.at[p],
                                     si)
                    pltpu.async_copy(dst_hbm.at[s, g + 2], dst_v.at[p], si)
            return 0
        lax.fori_loop(0, G, outer, 0)

        plsc.subcore_barrier()
        pltpu.sync_copy(acc_sh.at[pl.ds(s * ROWS_PT, ROWS_PT)],
                        z_hbm.at[c, pl.ds(s * ROWS_PT, ROWS_PT)])

    return agg


_agg128 = _make_agg(128)
_agg32 = _make_agg(32)


def _tc_first(xp, W1, degA, degB):
    def body(x_ref, w_ref, da_ref, db_ref, y_ref, dv_ref):
        dv = lax.rsqrt(1.0 + da_ref[...] + db_ref[...])
        xw = jnp.dot(x_ref[...].astype(jnp.bfloat16),
                     w_ref[...].astype(jnp.bfloat16),
                     preferred_element_type=jnp.float32)
        y = (xw * dv[:, None]).astype(jnp.bfloat16)
        y_ref[0] = y[:, :128]
        y_ref[1] = y[:, 128:]
        dv_ref[...] = dv

    return pl.pallas_call(
        body,
        grid=(GR,),
        in_specs=[
            pl.BlockSpec((BR, D_IN), lambda i: (i, 0)),
            pl.BlockSpec((D_IN, D_H), lambda i: (0, 0)),
            pl.BlockSpec((BR,), lambda i: (i,)),
            pl.BlockSpec((BR,), lambda i: (i,)),
        ],
        out_specs=[
            pl.BlockSpec((2, BR, 128), lambda i: (0, i, 0)),
            pl.BlockSpec((BR,), lambda i: (i,)),
        ],
        out_shape=[
            jax.ShapeDtypeStruct((2, NP, 128), jnp.bfloat16),
            jax.ShapeDtypeStruct((NP,), jnp.float32),
        ],
    )(xp, W1, degA, degB)


def _tc_mid(z, dinv, W, b, g, beta, d_out):
    dh = d_out // 2

    def body(z_ref, dv_ref, w_ref, b_ref, g_ref, bt_ref, y_ref):
        dv = dv_ref[...]
        zf = jnp.concatenate([z_ref[0], z_ref[1]],
                             axis=1).astype(jnp.float32)
        sg = g_ref[...] * _BN_S
        t = b_ref[...] * sg + bt_ref[...]
        h = jnp.maximum(zf * dv[:, None] * sg + t, 0.0)
        xw = jnp.dot(h.astype(jnp.bfloat16), w_ref[...].astype(jnp.bfloat16),
                     preferred_element_type=jnp.float32)
        y = (xw * dv[:, None]).astype(jnp.bfloat16)
        y_ref[0] = y[:, :dh]
        y_ref[1] = y[:, dh:]

    return pl.pallas_call(
        body,
        grid=(GR,),
        in_specs=[
            pl.BlockSpec((2, BR, 128), lambda i: (0, i, 0)),
            pl.BlockSpec((BR,), lambda i: (i,)),
            pl.BlockSpec((D_H, d_out), lambda i: (0, 0)),
            pl.BlockSpec((1, D_H), lambda i: (0, 0)),
            pl.BlockSpec((1, D_H), lambda i: (0, 0)),
            pl.BlockSpec((1, D_H), lambda i: (0, 0)),
        ],
        out_specs=pl.BlockSpec((2, BR, dh), lambda i: (0, i, 0)),
        out_shape=jax.ShapeDtypeStruct((2, NP, dh), jnp.bfloat16),
    )(z, dinv, W, b.reshape(1, -1), g.reshape(1, -1), beta.reshape(1, -1))


def _tc_softmax(z3, dinv, b3):
    def body(z_ref, dv_ref, b_ref, o_ref):
        zf = jnp.concatenate([z_ref[0], z_ref[1]],
                             axis=1).astype(jnp.float32)
        u = zf * dv_ref[...][:, None] + b_ref[...]
        m = jnp.max(u, axis=1, keepdims=True)
        e = jnp.exp(u - m)
        o_ref[...] = (u - m) - jnp.log(jnp.sum(e, axis=1, keepdims=True))

    return pl.pallas_call(
        body,
        grid=(GR,),
        in_specs=[
            pl.BlockSpec((2, BR, 32), lambda i: (0, i, 0)),
            pl.BlockSpec((BR,), lambda i: (i,)),
            pl.BlockSpec((1, D_OUT), lambda i: (0, 0)),
        ],
        out_specs=pl.BlockSpec((BR, D_OUT), lambda i: (i, 0)),
        out_shape=jax.ShapeDtypeStruct((NP, D_OUT), jnp.float32),
    )(z3, dinv, b3.reshape(1, -1))


def kernel(x, adj_t, W1, b1, g1, beta1, W2, b2, g2, beta2, W3, b3):
    src = adj_t[0]
    dst = adj_t[1]
    npad = EPAD - E
    padrows = N + (jnp.arange(npad, dtype=jnp.int32) % (NP - N))
    srcp = jnp.concatenate([src, padrows])
    dstp = jnp.concatenate([dst, padrows])
    src2 = srcp.reshape(NTILE, G, GC, CH)
    dstr = dstp.reshape(NTILE, G, GC, CH)
    degr = dstp.reshape(NCORE, NTILE, CW, CHD)
    xp = jnp.pad(x, ((0, NP - N), (0, 0)))

    deg = _deg_kernel(degr)
    y1, dinv = _tc_first(xp, W1, deg[0], deg[1])
    z1 = _agg128(y1, src2, dstr)
    y2 = _tc_mid(z1, dinv, W2, b1, g1, beta1, D_H)
    z2 = _agg128(y2, src2, dstr)
    y3 = _tc_mid(z2, dinv, W3, b2, g2, beta2, D_OUT)
    z3 = _agg32(y3, src2, dstr)
    out = _tc_softmax(z3, dinv, b3)
    return out[:N]
